# baseline (speedup 1.0000x reference)
import functools

import jax
import jax.numpy as jnp
from jax import lax
from jax.experimental import pallas as pl
from jax.experimental.pallas import tpu as pltpu
from jax.experimental.pallas import tpu_sc as plsc

N = 10000
E = 160000
D_IN = 256
D_OUT = 512
H = D_IN // 2
EPS = 1e-5

NC = 2
NS = 16
LANES = 16

EPT = E // NS
CHUNK = 80
NCHUNK = EPT // CHUNK

HP = 6144
NPASS = 2
REDH = HP // NS
DTOT = NPASS * HP

ROWS_PER_TILE = 624
ROWS_REM = N - NS * ROWS_PER_TILE
XCH = 80
NXF = ROWS_PER_TILE // XCH
XTAIL = ROWS_PER_TILE - NXF * XCH


def _rsqrt_newton(v):
    iv = plsc.bitcast(v, jnp.int32)
    y = plsc.bitcast(jnp.full((LANES,), 0x5F3759DF, jnp.int32) - (iv >> 1),
                     jnp.float32)
    for _ in range(3):
        y = y * (1.5 - 0.5 * v * y * y)
    return y


def _sc_body(x4_hbm, src_hbm, dst_hbm, dst4_hbm,
             agg_hbm, xs_hbm, dinv_hbm,
             idxbuf, d0b, d1b, rows0, rows1, hist, jbuf, accv, dloc,
             sem, gsem0, gsem1, ssem0, ssem1, dsem0, dsem1,
             shist, sdinv, accum):
    c = lax.axis_index("c")
    s = lax.axis_index("s")
    ones = jnp.full((LANES,), 1.0, jnp.float32)

    pltpu.async_copy(dst_hbm.at[s], idxbuf, sem).wait()

    for p in range(NPASS):
        lo = p * HP

        def zero(i, _):
            hist[pl.ds(i * LANES, LANES)] = jnp.zeros((LANES,), jnp.float32)
            return _
        lax.fori_loop(0, HP // LANES, zero, None)

        def scat(i, _):
            idx = idxbuf[pl.ds(i * LANES, LANES)]
            inb = (idx >= lo) & (idx < lo + HP)
            il = jnp.minimum(jnp.maximum(idx - lo, 0), HP - 1)
            plsc.addupdate_scatter(hist, [il], ones, mask=inb)
            return _
        lax.fori_loop(0, EPT // LANES, scat, None)

        pltpu.sync_copy(hist, shist.at[s])
        plsc.subcore_barrier()

        def zacc(k, _):
            accv[pl.ds(k * LANES, LANES)] = jnp.zeros((LANES,), jnp.float32)
            return _
        lax.fori_loop(0, REDH // LANES, zacc, None)

        def red_j(j, _):
            pltpu.sync_copy(shist.at[j, pl.ds(s * REDH, REDH)], jbuf)

            def red_k(k, __):
                sl = pl.ds(k * LANES, LANES)
                accv[sl] = accv[sl] + jbuf[sl]
                return __
            lax.fori_loop(0, REDH // LANES, red_k, None)
            return _
        lax.fori_loop(0, NS, red_j, None)

        def newt(k, _):
            sl = pl.ds(k * LANES, LANES)
            jbuf[sl] = _rsqrt_newton(accv[sl] + 1.0)
            return _
        lax.fori_loop(0, REDH // LANES, newt, None)

        pltpu.sync_copy(jbuf, sdinv.at[pl.ds(lo + s * REDH, REDH)])

        @pl.when(c == 0)
        def _():
            pltpu.sync_copy(jbuf, dinv_hbm.at[pl.ds(lo + s * REDH, REDH)])

        plsc.subcore_barrier()

    r0 = s * ROWS_PER_TILE
    pltpu.sync_copy(sdinv.at[pl.ds(r0, 640)], dloc)

    def xs_scale(buf, cnt, base):
        def sgroup(m, _):
            dvec = dloc[pl.ds(base + m * LANES, LANES)]
            for r in range(LANES):
                d = dvec[r]
                for k in range(H // LANES):
                    sl = pl.ds(k * LANES, LANES)
                    buf[m * LANES + r, sl] = buf[m * LANES + r, sl] * d
            return _
        lax.fori_loop(0, cnt // LANES, sgroup, None)

    def xs_in(j, buf, s_):
        pltpu.async_copy(x4_hbm.at[pl.ds(r0 + j * XCH, XCH), c], buf, s_)

    def xs_in_wait(j, buf, s_):
        pltpu.make_async_copy(x4_hbm.at[pl.ds(r0 + j * XCH, XCH), c], buf,
                              s_).wait()

    xbufs = (rows0, rows1)
    xsems = (gsem0, gsem1)
    xs_in(0, xbufs[0], xsems[0])
    for j in range(NXF):
        buf, s_ = xbufs[j % 2], xsems[j % 2]
        xs_in_wait(j, buf, s_)
        if j + 1 < NXF:
            xs_in(j + 1, xbufs[(j + 1) % 2], xsems[(j + 1) % 2])
        xs_scale(buf, XCH, j * XCH)
        row = r0 + j * XCH
        pltpu.sync_copy(buf, xs_hbm.at[c, pl.ds(row, XCH)])
        pltpu.sync_copy(buf, accum.at[pl.ds(row, XCH)])

    trow = r0 + NXF * XCH
    tbuf = xbufs[NXF % 2]
    pltpu.sync_copy(x4_hbm.at[pl.ds(trow, XTAIL), c], tbuf.at[pl.ds(0, XTAIL)])
    xs_scale(tbuf, XTAIL, NXF * XCH)
    pltpu.sync_copy(tbuf.at[pl.ds(0, XTAIL)], xs_hbm.at[c, pl.ds(trow, XTAIL)])
    pltpu.sync_copy(tbuf.at[pl.ds(0, XTAIL)], accum.at[pl.ds(trow, XTAIL)])

    @pl.when(s == NS - 1)
    def _():
        erow = NS * ROWS_PER_TILE
        pltpu.sync_copy(x4_hbm.at[pl.ds(erow, ROWS_REM), c],
                        rows0.at[pl.ds(0, ROWS_REM)])
        xs_scale(rows0, ROWS_REM, ROWS_PER_TILE)
        pltpu.sync_copy(rows0.at[pl.ds(0, ROWS_REM)],
                        xs_hbm.at[c, pl.ds(erow, ROWS_REM)])
        pltpu.sync_copy(rows0.at[pl.ds(0, ROWS_REM)],
                        accum.at[pl.ds(erow, ROWS_REM)])

    plsc.subcore_barrier()

    pltpu.async_copy(src_hbm.at[s], idxbuf, sem).wait()
    xs_c = xs_hbm.at[c]

    def src_ix(g):
        return idxbuf.at[pl.ds(g * CHUNK, CHUNK)]

    def dstart(g, b, s_):
        pltpu.async_copy(dst4_hbm.at[s, g], b, s_)

    def dwait(g, b, s_):
        pltpu.make_async_copy(dst4_hbm.at[s, g], b, s_).wait()

    def gather(g, buf, s_):
        pltpu.async_copy(xs_c.at[src_ix(g)], buf, s_)

    def gwait(g, buf, s_):
        pltpu.make_async_copy(xs_c.at[src_ix(g)], buf, s_).wait()

    def scat(buf, b, s_):
        pltpu.async_copy(buf, accum.at[b.at[0]], s_, add=True)

    def swait(buf, b, s_):
        pltpu.make_async_copy(buf, accum.at[b.at[0]], s_).wait()

    dstart(0, d0b, dsem0)
    dstart(1, d1b, dsem1)
    gather(0, rows0, gsem0)
    gwait(0, rows0, gsem0)
    gather(1, rows1, gsem1)
    dwait(0, d0b, dsem0)
    scat(rows0, d0b, ssem0)

    def step(i, _):
        g2 = 2 * i + 2
        g3 = jnp.minimum(2 * i + 3, NCHUNK - 1)
        gwait(2 * i + 1, rows1, gsem1)
        swait(rows0, d0b, ssem0)
        dstart(g2, d0b, dsem0)
        gather(g2, rows0, gsem0)
        dwait(2 * i + 1, d1b, dsem1)
        scat(rows1, d1b, ssem1)
        gwait(g2, rows0, gsem0)
        swait(rows1, d1b, ssem1)
        dstart(g3, d1b, dsem1)
        gather(g3, rows1, gsem1)
        dwait(g2, d0b, dsem0)
        scat(rows0, d0b, ssem0)
        return _
    lax.fori_loop(0, NCHUNK // 2, step, None)
    gwait(NCHUNK - 1, rows1, gsem1)
    dwait(NCHUNK - 1, d1b, dsem1)
    swait(rows0, d0b, ssem0)

    plsc.subcore_barrier()
    pltpu.sync_copy(accum.at[pl.ds(r0, ROWS_PER_TILE)],
                    agg_hbm.at[c, pl.ds(r0, ROWS_PER_TILE)])

    @pl.when(s == NS - 1)
    def _():
        erow = NS * ROWS_PER_TILE
        pltpu.sync_copy(accum.at[pl.ds(erow, ROWS_REM)],
                        agg_hbm.at[c, pl.ds(erow, ROWS_REM)])


def _make_sc_kernel():
    mesh = plsc.VectorSubcoreMesh(core_axis_name="c", subcore_axis_name="s")
    return pl.kernel(
        _sc_body,
        out_type=(
            jax.ShapeDtypeStruct((NC, N, H), jnp.float32),
            jax.ShapeDtypeStruct((NC, N, H), jnp.float32),
            jax.ShapeDtypeStruct((DTOT,), jnp.float32),
        ),
        mesh=mesh,
        scratch_types=[
            pltpu.VMEM((EPT,), jnp.int32),
            pltpu.VMEM((1, CHUNK), jnp.int32),
            pltpu.VMEM((1, CHUNK), jnp.int32),
            pltpu.VMEM((CHUNK, H), jnp.float32),
            pltpu.VMEM((CHUNK, H), jnp.float32),
            pltpu.VMEM((HP,), jnp.float32),
            pltpu.VMEM((REDH,), jnp.float32),
            pltpu.VMEM((REDH,), jnp.float32),
            pltpu.VMEM((640,), jnp.float32),
            pltpu.SemaphoreType.DMA,
            pltpu.SemaphoreType.DMA,
            pltpu.SemaphoreType.DMA,
            pltpu.SemaphoreType.DMA,
            pltpu.SemaphoreType.DMA,
            pltpu.SemaphoreType.DMA,
            pltpu.SemaphoreType.DMA,
            pltpu.VMEM_SHARED((NS, HP), jnp.float32),
            pltpu.VMEM_SHARED((DTOT,), jnp.float32),
            pltpu.VMEM_SHARED((N, H), jnp.float32),
        ],
        compiler_params=pltpu.CompilerParams(needs_layout_passes=False),
    )


RB = 2000
GRID = N // RB


def _mm_bn_body(agg_ref, dinv_ref, w_ref, b_ref, gamma_ref, beta_ref, y_ref,
                out_buf, stats_buf):
    p = pl.program_id(0)
    r = pl.program_id(1)

    @pl.when(p == 0)
    def _():
        aggf = jnp.concatenate([agg_ref[0], agg_ref[1]], axis=1)
        pre = aggf * dinv_ref[...]
        o = jnp.dot(pre, w_ref[...], preferred_element_type=jnp.float32)
        o = o + b_ref[...]
        out_buf[pl.ds(r * RB, RB), :] = o

        @pl.when(r == 0)
        def _():
            stats_buf[...] = jnp.zeros_like(stats_buf)

        stats_buf[0:1, :] += jnp.sum(o, axis=0, keepdims=True)
        stats_buf[1:2, :] += jnp.sum(o * o, axis=0, keepdims=True)

    @pl.when(p == 1)
    def _():
        mu = stats_buf[0:1, :] * (1.0 / N)
        ex2 = stats_buf[1:2, :] * (1.0 / N)
        var = jnp.maximum(ex2 - mu * mu, 0.0)
        inv = lax.rsqrt(var + EPS)
        o = out_buf[pl.ds(r * RB, RB), :]
        y = (o - mu) * (inv * gamma_ref[...]) + beta_ref[...]
        y_ref[...] = jnp.maximum(y, 0.0)


@jax.jit
def kernel(x, adj_t, W, b, gamma, beta):
    src = adj_t[0].astype(jnp.int32)
    dst = adj_t[1].astype(jnp.int32)

    agg, _xs, dinv = _make_sc_kernel()(
        x.reshape(N, NC, H),
        src.reshape(NS, EPT),
        dst.reshape(NS, EPT),
        dst.reshape(NS, NCHUNK, 1, CHUNK),
    )
    dinv2d = dinv[:N].reshape(N, 1)

    y = pl.pallas_call(
        _mm_bn_body,
        grid=(2, GRID),
        in_specs=[
            pl.BlockSpec((NC, RB, H), lambda p, r: (0, jnp.where(p == 0, r, 0), 0)),
            pl.BlockSpec((RB, 1), lambda p, r: (jnp.where(p == 0, r, 0), 0)),
            pl.BlockSpec((D_IN, D_OUT), lambda p, r: (0, 0)),
            pl.BlockSpec((1, D_OUT), lambda p, r: (0, 0)),
            pl.BlockSpec((1, D_OUT), lambda p, r: (0, 0)),
            pl.BlockSpec((1, D_OUT), lambda p, r: (0, 0)),
        ],
        out_specs=pl.BlockSpec((RB, D_OUT),
                               lambda p, r: (jnp.where(p == 0, 0, r), 0)),
        out_shape=jax.ShapeDtypeStruct((N, D_OUT), jnp.float32),
        scratch_shapes=[
            pltpu.VMEM((N, D_OUT), jnp.float32),
            pltpu.VMEM((2, D_OUT), jnp.float32),
        ],
    )(agg, dinv2d, W, b.reshape(1, D_OUT), gamma.reshape(1, D_OUT),
      beta.reshape(1, D_OUT))

    return y


# speedup vs baseline: 1.1014x; 1.1014x over previous
import functools

import jax
import jax.numpy as jnp
from jax import lax
from jax.experimental import pallas as pl
from jax.experimental.pallas import tpu as pltpu
from jax.experimental.pallas import tpu_sc as plsc

N = 10000
E = 160000
D_IN = 256
D_OUT = 512
H = D_IN // 2
EPS = 1e-5

NC = 2
NS = 16
LANES = 16

HPAD = 10240
EDGES_PER_TILE = E // NS
HIST_EDGES = E // (NC * NS) + 8
HIST_ITERS = HIST_EDGES // LANES
RED = HPAD // NS


def _deg_body(dst_hbm, degp_hbm, dstloc, hist, buf2, accv, sem, shared):
    c = lax.axis_index("c")
    s = lax.axis_index("s")

    pltpu.async_copy(dst_hbm.at[c, s], dstloc, sem).wait()

    def zero(i, _):
        hist[pl.ds(i * LANES, LANES)] = jnp.zeros((LANES,), jnp.float32)
        return _
    lax.fori_loop(0, HPAD // LANES, zero, None)

    ones = jnp.full((LANES,), 1.0, jnp.float32)

    def scat(i, _):
        idx = dstloc[pl.ds(i * LANES, LANES)]
        plsc.addupdate_scatter(hist, [idx], ones)
        return _
    lax.fori_loop(0, HIST_ITERS, scat, None)

    pltpu.sync_copy(hist, shared.at[s])
    plsc.subcore_barrier()
    pltpu.sync_copy(shared.at[:, pl.ds(s * RED, RED)], buf2)

    def zacc(k, _):
        accv[pl.ds(k * LANES, LANES)] = jnp.zeros((LANES,), jnp.float32)
        return _
    lax.fori_loop(0, RED // LANES, zacc, None)

    def red_j(j, _):
        def red_k(k, __):
            sl = pl.ds(k * LANES, LANES)
            accv[sl] = accv[sl] + buf2[j, sl]
            return __
        lax.fori_loop(0, RED // LANES, red_k, None)
        return _
    lax.fori_loop(0, NS, red_j, None)

    pltpu.sync_copy(accv, degp_hbm.at[c, pl.ds(s * RED, RED)])


def _make_deg_kernel():
    mesh = plsc.VectorSubcoreMesh(core_axis_name="c", subcore_axis_name="s")

    return pl.kernel(
        _deg_body,
        out_type=jax.ShapeDtypeStruct((NC, HPAD), jnp.float32),
        mesh=mesh,
        scratch_types=[
            pltpu.VMEM((HIST_EDGES,), jnp.int32),
            pltpu.VMEM((HPAD,), jnp.float32),
            pltpu.VMEM((NS, RED), jnp.float32),
            pltpu.VMEM((RED,), jnp.float32),
            pltpu.SemaphoreType.DMA,
            pltpu.VMEM_SHARED((NS, HPAD), jnp.float32),
        ],
        compiler_params=pltpu.CompilerParams(needs_layout_passes=False),
    )


CHUNK = 80
NCHUNK = E // (NS * CHUNK)
ROWS_PER_TILE = 624
ROWS_REM = N - NS * ROWS_PER_TILE


def _make_agg_kernel():
    mesh = plsc.VectorSubcoreMesh(core_axis_name="c", subcore_axis_name="s")

    def body(xs_hbm, src_hbm, dst_hbm, agg_hbm, srcloc, dstloc, rows0, rows1,
             sem, sem0, sem1, ssem0, ssem1, accum):
        c = lax.axis_index("c")
        s = lax.axis_index("s")

        pltpu.async_copy(src_hbm.at[s], srcloc, sem).wait()
        pltpu.async_copy(dst_hbm.at[s], dstloc, sem).wait()

        r0 = s * ROWS_PER_TILE
        pltpu.sync_copy(xs_hbm.at[c, pl.ds(r0, ROWS_PER_TILE)],
                        accum.at[pl.ds(r0, ROWS_PER_TILE)])

        @pl.when(s == NS - 1)
        def _():
            pltpu.sync_copy(xs_hbm.at[c, pl.ds(NS * ROWS_PER_TILE, ROWS_REM)],
                            accum.at[pl.ds(NS * ROWS_PER_TILE, ROWS_REM)])

        plsc.subcore_barrier()

        xs_c = xs_hbm.at[c]

        def src_ix(g):
            return srcloc.at[pl.ds(g * CHUNK, CHUNK)]

        def gather(g, buf, s_):
            pltpu.async_copy(xs_c.at[src_ix(g)], buf, s_)

        def gwait(g, buf, s_):
            pltpu.make_async_copy(xs_c.at[src_ix(g)], buf, s_).wait()

        def scat(g, buf, s_):
            pltpu.async_copy(buf, accum.at[dstloc.at[g]], s_, add=True)

        def swait(g, buf, s_):
            pltpu.make_async_copy(buf, accum.at[dstloc.at[g]], s_).wait()

        gather(0, rows0, sem0)
        gwait(0, rows0, sem0)
        gather(1, rows1, sem1)
        scat(0, rows0, ssem0)

        def step(i, _):
            g1 = 2 * i + 1
            g2 = 2 * i + 2
            g3 = jnp.minimum(2 * i + 3, NCHUNK - 1)
            gwait(g1, rows1, sem1)
            swait(g1 - 1, rows0, ssem0)
            gather(g2, rows0, sem0)
            scat(g1, rows1, ssem1)
            gwait(g2, rows0, sem0)
            swait(g1, rows1, ssem1)
            gather(g3, rows1, sem1)
            scat(g2, rows0, ssem0)
            return _
        lax.fori_loop(0, NCHUNK // 2, step, None)
        gwait(NCHUNK - 1, rows1, sem1)
        swait(NCHUNK - 1, rows0, ssem0)

        plsc.subcore_barrier()
        pltpu.sync_copy(accum.at[pl.ds(r0, ROWS_PER_TILE)],
                        agg_hbm.at[c, pl.ds(r0, ROWS_PER_TILE)])

        @pl.when(s == NS - 1)
        def _():
            pltpu.sync_copy(accum.at[pl.ds(NS * ROWS_PER_TILE, ROWS_REM)],
                            agg_hbm.at[c, pl.ds(NS * ROWS_PER_TILE, ROWS_REM)])

    return pl.kernel(
        body,
        out_type=jax.ShapeDtypeStruct((NC, N, H), jnp.float32),
        mesh=mesh,
        scratch_types=[
            pltpu.VMEM((EDGES_PER_TILE,), jnp.int32),
            pltpu.VMEM((NCHUNK, CHUNK), jnp.int32),
            pltpu.VMEM((CHUNK, H), jnp.float32),
            pltpu.VMEM((CHUNK, H), jnp.float32),
            pltpu.SemaphoreType.DMA,
            pltpu.SemaphoreType.DMA,
            pltpu.SemaphoreType.DMA,
            pltpu.SemaphoreType.DMA,
            pltpu.SemaphoreType.DMA,
            pltpu.VMEM_SHARED((N, H), jnp.float32),
        ],
    )


RB = 2000
GRID = N // RB


def _prescale_body(d0_ref, d1_ref, x_ref, dinv_ref, xs_ref):
    d = d0_ref[...] + d1_ref[...] + 1.0
    dinv = jnp.where(d > 0.0, lax.rsqrt(d), 0.0)
    dinv_ref[...] = dinv
    xsb = x_ref[...] * dinv
    xs_ref[0] = xsb[:, :H]
    xs_ref[1] = xsb[:, H:]


def _mm_bn_body(agg_ref, dinv_ref, w_ref, b_ref, gamma_ref, beta_ref, y_ref,
                out_buf, stats_buf):
    p = pl.program_id(0)
    r = pl.program_id(1)

    @pl.when(p == 0)
    def _():
        aggf = jnp.concatenate([agg_ref[0], agg_ref[1]], axis=1)
        pre = aggf * dinv_ref[...]
        o = jnp.dot(pre, w_ref[...], preferred_element_type=jnp.float32)
        o = o + b_ref[...]
        out_buf[pl.ds(r * RB, RB), :] = o

        @pl.when(r == 0)
        def _():
            stats_buf[...] = jnp.zeros_like(stats_buf)

        stats_buf[0:1, :] += jnp.sum(o, axis=0, keepdims=True)
        stats_buf[1:2, :] += jnp.sum(o * o, axis=0, keepdims=True)

    @pl.when(p == 1)
    def _():
        mu = stats_buf[0:1, :] * (1.0 / N)
        ex2 = stats_buf[1:2, :] * (1.0 / N)
        var = jnp.maximum(ex2 - mu * mu, 0.0)
        inv = lax.rsqrt(var + EPS)
        o = out_buf[pl.ds(r * RB, RB), :]
        y = (o - mu) * (inv * gamma_ref[...]) + beta_ref[...]
        y_ref[...] = jnp.maximum(y, 0.0)


@jax.jit
def kernel(x, adj_t, W, b, gamma, beta):
    src = adj_t[0].astype(jnp.int32)
    dst = adj_t[1].astype(jnp.int32)

    dst_h = jnp.pad(dst.reshape(NC, NS, E // (NC * NS)),
                    ((0, 0), (0, 0), (0, 8)), constant_values=N)
    degp = _make_deg_kernel()(dst_h)
    d0 = degp[0, :N].reshape(N, 1)
    d1 = degp[1, :N].reshape(N, 1)

    dinv2d, xs = pl.pallas_call(
        _prescale_body,
        grid=(GRID,),
        in_specs=[
            pl.BlockSpec((RB, 1), lambda r: (r, 0)),
            pl.BlockSpec((RB, 1), lambda r: (r, 0)),
            pl.BlockSpec((RB, D_IN), lambda r: (r, 0)),
        ],
        out_specs=[
            pl.BlockSpec((RB, 1), lambda r: (r, 0)),
            pl.BlockSpec((NC, RB, H), lambda r: (0, r, 0)),
        ],
        out_shape=[
            jax.ShapeDtypeStruct((N, 1), jnp.float32),
            jax.ShapeDtypeStruct((NC, N, H), jnp.float32),
        ],
    )(d0, d1, x)

    agg = _make_agg_kernel()(
        xs,
        src.reshape(NS, EDGES_PER_TILE),
        dst.reshape(NS, NCHUNK, CHUNK),
    )

    y = pl.pallas_call(
        _mm_bn_body,
        grid=(2, GRID),
        in_specs=[
            pl.BlockSpec((NC, RB, H), lambda p, r: (0, jnp.where(p == 0, r, 0), 0)),
            pl.BlockSpec((RB, 1), lambda p, r: (jnp.where(p == 0, r, 0), 0)),
            pl.BlockSpec((D_IN, D_OUT), lambda p, r: (0, 0)),
            pl.BlockSpec((1, D_OUT), lambda p, r: (0, 0)),
            pl.BlockSpec((1, D_OUT), lambda p, r: (0, 0)),
            pl.BlockSpec((1, D_OUT), lambda p, r: (0, 0)),
        ],
        out_specs=pl.BlockSpec((RB, D_OUT),
                               lambda p, r: (jnp.where(p == 0, 0, r), 0)),
        out_shape=jax.ShapeDtypeStruct((N, D_OUT), jnp.float32),
        scratch_shapes=[
            pltpu.VMEM((N, D_OUT), jnp.float32),
            pltpu.VMEM((2, D_OUT), jnp.float32),
        ],
    )(agg, dinv2d, W, b.reshape(1, D_OUT), gamma.reshape(1, D_OUT),
      beta.reshape(1, D_OUT))

    return y


# speedup vs baseline: 1.1029x; 1.0014x over previous
import functools

import jax
import jax.numpy as jnp
from jax import lax
from jax.experimental import pallas as pl
from jax.experimental.pallas import tpu as pltpu
from jax.experimental.pallas import tpu_sc as plsc

N = 10000
E = 160000
D_IN = 256
D_OUT = 512
H = D_IN // 2
EPS = 1e-5

NC = 2
NS = 16
LANES = 16

HPAD = 10240
EDGES_PER_TILE = E // NS
HIST_EDGES = E // (NC * NS) + 8
HIST_ITERS = HIST_EDGES // LANES
RED = HPAD // NS


def _deg_body(dst_hbm, degp_hbm, dstloc, hist, buf2, accv, sem, shared):
    c = lax.axis_index("c")
    s = lax.axis_index("s")

    pltpu.async_copy(dst_hbm.at[c, s], dstloc, sem).wait()

    def zero(i, _):
        hist[pl.ds(i * LANES, LANES)] = jnp.zeros((LANES,), jnp.float32)
        return _
    lax.fori_loop(0, HPAD // LANES, zero, None)

    ones = jnp.full((LANES,), 1.0, jnp.float32)

    def scat(i, _):
        idx = dstloc[pl.ds(i * LANES, LANES)]
        plsc.addupdate_scatter(hist, [idx], ones)
        return _
    lax.fori_loop(0, HIST_ITERS, scat, None)

    pltpu.sync_copy(hist, shared.at[s])
    plsc.subcore_barrier()
    pltpu.sync_copy(shared.at[:, pl.ds(s * RED, RED)], buf2)

    def zacc(k, _):
        accv[pl.ds(k * LANES, LANES)] = jnp.zeros((LANES,), jnp.float32)
        return _
    lax.fori_loop(0, RED // LANES, zacc, None)

    def red_j(j, _):
        def red_k(k, __):
            sl = pl.ds(k * LANES, LANES)
            accv[sl] = accv[sl] + buf2[j, sl]
            return __
        lax.fori_loop(0, RED // LANES, red_k, None)
        return _
    lax.fori_loop(0, NS, red_j, None)

    pltpu.sync_copy(accv, degp_hbm.at[c, pl.ds(s * RED, RED)])


def _make_deg_kernel():
    mesh = plsc.VectorSubcoreMesh(core_axis_name="c", subcore_axis_name="s")

    return pl.kernel(
        _deg_body,
        out_type=jax.ShapeDtypeStruct((NC, HPAD), jnp.float32),
        mesh=mesh,
        scratch_types=[
            pltpu.VMEM((HIST_EDGES,), jnp.int32),
            pltpu.VMEM((HPAD,), jnp.float32),
            pltpu.VMEM((NS, RED), jnp.float32),
            pltpu.VMEM((RED,), jnp.float32),
            pltpu.SemaphoreType.DMA,
            pltpu.VMEM_SHARED((NS, HPAD), jnp.float32),
        ],
        compiler_params=pltpu.CompilerParams(needs_layout_passes=False),
    )


CHUNK = 80
NCHUNK = E // (NS * CHUNK)
ROWS_PER_TILE = 624
ROWS_REM = N - NS * ROWS_PER_TILE


def _make_agg_kernel():
    mesh = plsc.VectorSubcoreMesh(core_axis_name="c", subcore_axis_name="s")

    def body(xs_hbm, src_hbm, dst_hbm, agg_hbm, srcloc, dstloc, rows0, rows1,
             sem, sem0, sem1, ssem0, ssem1, accum):
        c = lax.axis_index("c")
        s = lax.axis_index("s")

        pltpu.async_copy(src_hbm.at[s], srcloc, sem).wait()
        pltpu.async_copy(dst_hbm.at[s], dstloc, sem).wait()

        r0 = s * ROWS_PER_TILE
        pltpu.sync_copy(xs_hbm.at[c, pl.ds(r0, ROWS_PER_TILE)],
                        accum.at[pl.ds(r0, ROWS_PER_TILE)])

        @pl.when(s == NS - 1)
        def _():
            pltpu.sync_copy(xs_hbm.at[c, pl.ds(NS * ROWS_PER_TILE, ROWS_REM)],
                            accum.at[pl.ds(NS * ROWS_PER_TILE, ROWS_REM)])

        plsc.subcore_barrier()

        xs_c = xs_hbm.at[c]

        def src_ix(g):
            return srcloc.at[pl.ds(g * CHUNK, CHUNK)]

        def gather(g, buf, s_):
            pltpu.async_copy(xs_c.at[src_ix(g)], buf, s_)

        def gwait(g, buf, s_):
            pltpu.make_async_copy(xs_c.at[src_ix(g)], buf, s_).wait()

        def scat(g, buf, s_):
            pltpu.async_copy(buf, accum.at[dstloc.at[g]], s_, add=True)

        def swait(g, buf, s_):
            pltpu.make_async_copy(buf, accum.at[dstloc.at[g]], s_).wait()

        gather(0, rows0, sem0)
        gwait(0, rows0, sem0)
        gather(1, rows1, sem1)
        scat(0, rows0, ssem0)

        def step(i, _):
            g1 = 2 * i + 1
            g2 = 2 * i + 2
            g3 = jnp.minimum(2 * i + 3, NCHUNK - 1)
            gwait(g1, rows1, sem1)
            swait(g1 - 1, rows0, ssem0)
            gather(g2, rows0, sem0)
            scat(g1, rows1, ssem1)
            gwait(g2, rows0, sem0)
            swait(g1, rows1, ssem1)
            gather(g3, rows1, sem1)
            scat(g2, rows0, ssem0)
            return _
        lax.fori_loop(0, NCHUNK // 2, step, None)
        gwait(NCHUNK - 1, rows1, sem1)
        swait(NCHUNK - 1, rows0, ssem0)

        plsc.subcore_barrier()
        pltpu.sync_copy(accum.at[pl.ds(r0, ROWS_PER_TILE)],
                        agg_hbm.at[c, pl.ds(r0, ROWS_PER_TILE)])

        @pl.when(s == NS - 1)
        def _():
            pltpu.sync_copy(accum.at[pl.ds(NS * ROWS_PER_TILE, ROWS_REM)],
                            agg_hbm.at[c, pl.ds(NS * ROWS_PER_TILE, ROWS_REM)])

    return pl.kernel(
        body,
        out_type=jax.ShapeDtypeStruct((NC, N, H), jnp.float32),
        mesh=mesh,
        scratch_types=[
            pltpu.VMEM((EDGES_PER_TILE,), jnp.int32),
            pltpu.VMEM((NCHUNK, CHUNK), jnp.int32),
            pltpu.VMEM((CHUNK, H), jnp.float32),
            pltpu.VMEM((CHUNK, H), jnp.float32),
            pltpu.SemaphoreType.DMA,
            pltpu.SemaphoreType.DMA,
            pltpu.SemaphoreType.DMA,
            pltpu.SemaphoreType.DMA,
            pltpu.SemaphoreType.DMA,
            pltpu.VMEM_SHARED((N, H), jnp.float32),
        ],
    )


RB = 2000
GRID = N // RB


def _prescale_body(d0_ref, d1_ref, x_ref, dinv_ref, xs_ref):
    d = d0_ref[...] + d1_ref[...] + 1.0
    dinv = jnp.where(d > 0.0, lax.rsqrt(d), 0.0)
    dinv_ref[...] = dinv
    xsb = x_ref[...] * dinv
    xs_ref[0] = xsb[:, :H]
    xs_ref[1] = xsb[:, H:]


def _mm_bn_body(agg_ref, dinv_ref, w_ref, b_ref, gamma_ref, beta_ref, y_ref,
                out_buf, stats_buf):
    p = pl.program_id(0)
    r = pl.program_id(1)

    @pl.when(p == 0)
    def _():
        aggf = jnp.concatenate([agg_ref[0], agg_ref[1]], axis=1)
        pre = (aggf * dinv_ref[...]).astype(jnp.bfloat16)
        o = jnp.dot(pre, w_ref[...], preferred_element_type=jnp.float32)
        o = o + b_ref[...]
        out_buf[pl.ds(r * RB, RB), :] = o

        @pl.when(r == 0)
        def _():
            stats_buf[...] = jnp.zeros_like(stats_buf)

        stats_buf[0:1, :] += jnp.sum(o, axis=0, keepdims=True)
        stats_buf[1:2, :] += jnp.sum(o * o, axis=0, keepdims=True)

    @pl.when(p == 1)
    def _():
        mu = stats_buf[0:1, :] * (1.0 / N)
        ex2 = stats_buf[1:2, :] * (1.0 / N)
        var = jnp.maximum(ex2 - mu * mu, 0.0)
        inv = lax.rsqrt(var + EPS)
        o = out_buf[pl.ds(r * RB, RB), :]
        y = (o - mu) * (inv * gamma_ref[...]) + beta_ref[...]
        y_ref[...] = jnp.maximum(y, 0.0)


@jax.jit
def kernel(x, adj_t, W, b, gamma, beta):
    src = adj_t[0].astype(jnp.int32)
    dst = adj_t[1].astype(jnp.int32)

    dst_h = jnp.pad(dst.reshape(NC, NS, E // (NC * NS)),
                    ((0, 0), (0, 0), (0, 8)), constant_values=N)
    degp = _make_deg_kernel()(dst_h)
    d0 = degp[0, :N].reshape(N, 1)
    d1 = degp[1, :N].reshape(N, 1)

    dinv2d, xs = pl.pallas_call(
        _prescale_body,
        grid=(GRID,),
        in_specs=[
            pl.BlockSpec((RB, 1), lambda r: (r, 0)),
            pl.BlockSpec((RB, 1), lambda r: (r, 0)),
            pl.BlockSpec((RB, D_IN), lambda r: (r, 0)),
        ],
        out_specs=[
            pl.BlockSpec((RB, 1), lambda r: (r, 0)),
            pl.BlockSpec((NC, RB, H), lambda r: (0, r, 0)),
        ],
        out_shape=[
            jax.ShapeDtypeStruct((N, 1), jnp.float32),
            jax.ShapeDtypeStruct((NC, N, H), jnp.float32),
        ],
    )(d0, d1, x)

    agg = _make_agg_kernel()(
        xs,
        src.reshape(NS, EDGES_PER_TILE),
        dst.reshape(NS, NCHUNK, CHUNK),
    )

    y = pl.pallas_call(
        _mm_bn_body,
        grid=(2, GRID),
        in_specs=[
            pl.BlockSpec((NC, RB, H), lambda p, r: (0, jnp.where(p == 0, r, 0), 0)),
            pl.BlockSpec((RB, 1), lambda p, r: (jnp.where(p == 0, r, 0), 0)),
            pl.BlockSpec((D_IN, D_OUT), lambda p, r: (0, 0)),
            pl.BlockSpec((1, D_OUT), lambda p, r: (0, 0)),
            pl.BlockSpec((1, D_OUT), lambda p, r: (0, 0)),
            pl.BlockSpec((1, D_OUT), lambda p, r: (0, 0)),
        ],
        out_specs=pl.BlockSpec((RB, D_OUT),
                               lambda p, r: (jnp.where(p == 0, 0, r), 0)),
        out_shape=jax.ShapeDtypeStruct((N, D_OUT), jnp.float32),
        scratch_shapes=[
            pltpu.VMEM((N, D_OUT), jnp.float32),
            pltpu.VMEM((2, D_OUT), jnp.float32),
        ],
    )(agg, dinv2d, W.astype(jnp.bfloat16), b.reshape(1, D_OUT),
      gamma.reshape(1, D_OUT),
      beta.reshape(1, D_OUT))

    return y
